# Initial kernel scaffold; baseline (speedup 1.0000x reference)
import jax
import jax.numpy as jnp
from jax.experimental import pallas as pl


def kernel(x, edge_index, edge_attr, u, batch, W1, b1, W2, b2):
    raise NotImplementedError("write your pallas kernel here")


# speedup vs baseline: 4.5760x; 4.5760x over previous
import functools

import jax
import jax.numpy as jnp
from jax import lax
from jax.experimental import pallas as pl
from jax.experimental.pallas import tpu as pltpu
from jax.experimental.pallas import tpu_sc as plsc

_NC, _NS = 2, 16
_NW = _NC * _NS
_N = 100000
_D = 128
_G = 256
_BLK = 128
_NFULL = _N // _BLK
_TAIL = _N - _NFULL * _BLK
_NBLK = _NFULL + (1 if _TAIL else 0)
_ITERS = -(-_NBLK // _NW)
_GR = _G // _NS


def _sc_segment_sums(x, batch_i32):
    mesh = plsc.VectorSubcoreMesh(
        core_axis_name="c", subcore_axis_name="s",
        num_cores=_NC, num_subcores=_NS)

    @functools.partial(
        pl.kernel,
        out_type=(
            jax.ShapeDtypeStruct((_NC, _G, _D), jnp.float32),
            jax.ShapeDtypeStruct((_NC, _G, _D), jnp.float32),
        ),
        mesh=mesh,
        scratch_types=[
            pltpu.VMEM((_BLK, _D), jnp.float32),
            pltpu.VMEM((_BLK,), jnp.int32),
            pltpu.VMEM((_TAIL, _D), jnp.float32),
            pltpu.VMEM((_TAIL,), jnp.int32),
            pltpu.VMEM((_BLK, _D), jnp.float32),
            pltpu.VMEM((_GR, _D), jnp.float32),
            pltpu.VMEM_SHARED((_G, _D), jnp.float32),
            pltpu.VMEM_SHARED((_G, _D), jnp.float32),
        ],
    )
    def k(x_hbm, b_hbm, sums_out, cnt_out,
          xb, idx, xbt, idxt, ones_v, zb, acc, cacc):
        cid = lax.axis_index("c")
        sid = lax.axis_index("s")
        wid = sid * _NC + cid

        zeros16 = jnp.zeros((16,), jnp.float32)
        ones16 = jnp.ones((16,), jnp.float32)
        for i in range(_GR):
            for j in range(_D // 16):
                zb[i, pl.ds(j * 16, 16)] = zeros16
        for i in range(_BLK):
            for j in range(_D // 16):
                ones_v[i, pl.ds(j * 16, 16)] = ones16

        pltpu.sync_copy(zb, acc.at[pl.ds(sid * _GR, _GR)])
        pltpu.sync_copy(zb, cacc.at[pl.ds(sid * _GR, _GR)])
        plsc.subcore_barrier()

        for it in range(_ITERS):
            blk = wid + it * _NW

            @pl.when(blk < _NFULL)
            def _():
                base = blk * _BLK
                pltpu.sync_copy(b_hbm.at[pl.ds(base, _BLK)], idx)
                pltpu.sync_copy(x_hbm.at[pl.ds(base, _BLK)], xb)
                pltpu.sync_copy(xb, acc.at[idx], add=True)
                pltpu.sync_copy(ones_v, cacc.at[idx], add=True)

            if _TAIL:
                @pl.when(blk == _NFULL)
                def _():
                    base = _NFULL * _BLK
                    pltpu.sync_copy(b_hbm.at[pl.ds(base, _TAIL)], idxt)
                    pltpu.sync_copy(x_hbm.at[pl.ds(base, _TAIL)], xbt)
                    pltpu.sync_copy(xbt, acc.at[idxt], add=True)
                    pltpu.sync_copy(ones_v.at[pl.ds(0, _TAIL)],
                                    cacc.at[idxt], add=True)

        plsc.subcore_barrier()

        @pl.when(sid == 0)
        def _():
            pltpu.sync_copy(acc, sums_out.at[cid])
            pltpu.sync_copy(cacc, cnt_out.at[cid])

    return k(x, batch_i32)


def _mlp(u, sums2, cnt2, w1u, w1x, b1, w2, b2):
    def body(u_ref, s_ref, c_ref, w1u_ref, w1x_ref, b1_ref, w2_ref, b2_ref,
             o_ref):
        sums = s_ref[0] + s_ref[1]
        cnt = c_ref[0, :, 0:1] + c_ref[1, :, 0:1]
        mean = sums / jnp.maximum(cnt, 1.0)
        h = (jnp.dot(u_ref[...], w1u_ref[...],
                     preferred_element_type=jnp.float32)
             + jnp.dot(mean, w1x_ref[...], preferred_element_type=jnp.float32)
             + b1_ref[...])
        h = jnp.maximum(h, 0.0)
        o_ref[...] = (jnp.dot(h, w2_ref[...],
                              preferred_element_type=jnp.float32)
                      + b2_ref[...])

    return pl.pallas_call(
        body,
        out_shape=jax.ShapeDtypeStruct((_G, 32), jnp.float32),
    )(u, sums2, cnt2, w1u, w1x, b1, w2, b2)


def kernel(x, edge_index, edge_attr, u, batch, W1, b1, W2, b2):
    del edge_index, edge_attr
    b32 = batch.astype(jnp.int32)
    sums2, cnt2 = _sc_segment_sums(x, b32)
    n_glob = u.shape[1]
    return _mlp(u, sums2, cnt2, W1[:n_glob], W1[n_glob:],
                b1.reshape(1, -1), W2, b2.reshape(1, -1))


# speedup vs baseline: 6.3570x; 1.3892x over previous
import functools

import jax
import jax.numpy as jnp
from jax import lax
from jax.experimental import pallas as pl
from jax.experimental.pallas import tpu as pltpu
from jax.experimental.pallas import tpu_sc as plsc

_NC, _NS = 2, 16
_NW = _NC * _NS
_N = 100000
_D = 128
_G = 256
_BLK = 128
_NFULL = _N // _BLK
_TAIL = _N - _NFULL * _BLK
_NBLK = _NFULL + (1 if _TAIL else 0)
_ITERS = -(-_NBLK // _NW)
_STEADY = _NFULL // _NW
_GR = _G // _NS


def _sc_segment_sums(x, batch_i32):
    mesh = plsc.VectorSubcoreMesh(
        core_axis_name="c", subcore_axis_name="s",
        num_cores=_NC, num_subcores=_NS)

    @functools.partial(
        pl.kernel,
        out_type=(
            jax.ShapeDtypeStruct((_NC, _G, _D), jnp.float32),
            jax.ShapeDtypeStruct((_NC, _G, _D), jnp.float32),
        ),
        mesh=mesh,
        scratch_types=[
            pltpu.VMEM((2, _BLK, _D), jnp.float32),
            pltpu.VMEM((2, _BLK), jnp.int32),
            pltpu.VMEM((_TAIL, _D), jnp.float32),
            pltpu.VMEM((_TAIL,), jnp.int32),
            pltpu.VMEM((_BLK, _D), jnp.float32),
            pltpu.VMEM((_GR, _D), jnp.float32),
            pltpu.VMEM_SHARED((_G, _D), jnp.float32),
            pltpu.VMEM_SHARED((_G, _D), jnp.float32),
            pltpu.SemaphoreType.DMA,
            pltpu.SemaphoreType.DMA,
            pltpu.SemaphoreType.DMA,
            pltpu.SemaphoreType.DMA,
        ],
    )
    def k(x_hbm, b_hbm, sums_out, cnt_out,
          xb, idx, xbt, idxt, ones_v, zb, acc, cacc,
          lx0, li0, lx1, li1):
        cid = lax.axis_index("c")
        sid = lax.axis_index("s")
        wid = sid * _NC + cid
        lsem = ((lx0, li0), (lx1, li1))

        zeros16 = jnp.zeros((16,), jnp.float32)
        ones16 = jnp.ones((16,), jnp.float32)
        for i in range(_GR):
            for j in range(_D // 16):
                zb[i, pl.ds(j * 16, 16)] = zeros16
        for i in range(_BLK):
            for j in range(_D // 16):
                ones_v[i, pl.ds(j * 16, 16)] = ones16

        pltpu.sync_copy(zb, acc.at[pl.ds(sid * _GR, _GR)])
        pltpu.sync_copy(zb, cacc.at[pl.ds(sid * _GR, _GR)])
        plsc.subcore_barrier()

        def issue_loads(it, p):
            base = (wid + it * _NW) * _BLK
            dx = pltpu.async_copy(x_hbm.at[pl.ds(base, _BLK)],
                                  xb.at[p], lsem[p][0])
            di = pltpu.async_copy(b_hbm.at[pl.ds(base, _BLK)],
                                  idx.at[p], lsem[p][1])
            return dx, di

        pend_ld = issue_loads(0, 0)
        for it in range(_STEADY):
            p = it & 1
            for dsc in pend_ld:
                dsc.wait()
            if it + 1 < _STEADY:
                pend_ld = issue_loads(it + 1, 1 - p)
            pltpu.sync_copy(xb.at[p], acc.at[idx.at[p]], add=True)
            pltpu.sync_copy(ones_v, cacc.at[idx.at[p]], add=True)

        for it in range(_STEADY, _ITERS):
            blk = wid + it * _NW

            @pl.when(blk < _NFULL)
            def _():
                base = blk * _BLK
                pltpu.sync_copy(b_hbm.at[pl.ds(base, _BLK)], idx.at[0])
                pltpu.sync_copy(x_hbm.at[pl.ds(base, _BLK)], xb.at[0])
                pltpu.sync_copy(xb.at[0], acc.at[idx.at[0]], add=True)
                pltpu.sync_copy(ones_v, cacc.at[idx.at[0]], add=True)

            if _TAIL:
                @pl.when(blk == _NFULL)
                def _():
                    base = _NFULL * _BLK
                    pltpu.sync_copy(b_hbm.at[pl.ds(base, _TAIL)], idxt)
                    pltpu.sync_copy(x_hbm.at[pl.ds(base, _TAIL)], xbt)
                    pltpu.sync_copy(xbt, acc.at[idxt], add=True)
                    pltpu.sync_copy(ones_v.at[pl.ds(0, _TAIL)],
                                    cacc.at[idxt], add=True)

        plsc.subcore_barrier()

        @pl.when(sid == 0)
        def _():
            pltpu.sync_copy(acc, sums_out.at[cid])
            pltpu.sync_copy(cacc, cnt_out.at[cid])

    return k(x, batch_i32)


def _mlp(u, sums2, cnt2, w1u, w1x, b1, w2, b2):
    def body(u_ref, s_ref, c_ref, w1u_ref, w1x_ref, b1_ref, w2_ref, b2_ref,
             o_ref):
        sums = s_ref[0] + s_ref[1]
        cnt = c_ref[0, :, 0:1] + c_ref[1, :, 0:1]
        mean = sums / jnp.maximum(cnt, 1.0)
        h = (jnp.dot(u_ref[...], w1u_ref[...],
                     preferred_element_type=jnp.float32)
             + jnp.dot(mean, w1x_ref[...], preferred_element_type=jnp.float32)
             + b1_ref[...])
        h = jnp.maximum(h, 0.0)
        o_ref[...] = (jnp.dot(h, w2_ref[...],
                              preferred_element_type=jnp.float32)
                      + b2_ref[...])

    return pl.pallas_call(
        body,
        out_shape=jax.ShapeDtypeStruct((_G, 32), jnp.float32),
    )(u, sums2, cnt2, w1u, w1x, b1, w2, b2)


def kernel(x, edge_index, edge_attr, u, batch, W1, b1, W2, b2):
    del edge_index, edge_attr
    b32 = batch.astype(jnp.int32)
    sums2, cnt2 = _sc_segment_sums(x, b32)
    n_glob = u.shape[1]
    return _mlp(u, sums2, cnt2, W1[:n_glob], W1[n_glob:],
                b1.reshape(1, -1), W2, b2.reshape(1, -1))


# speedup vs baseline: 6.4208x; 1.0100x over previous
import functools

import jax
import jax.numpy as jnp
from jax import lax
from jax.experimental import pallas as pl
from jax.experimental.pallas import tpu as pltpu
from jax.experimental.pallas import tpu_sc as plsc

_NC, _NS = 2, 16
_NW = _NC * _NS
_N = 100000
_D = 128
_G = 256
_BLK = 128
_NFULL = _N // _BLK
_TAIL = _N - _NFULL * _BLK
_NBLK = _NFULL + (1 if _TAIL else 0)
_ITERS = -(-_NBLK // _NW)
_STEADY = _NFULL // _NW
_GR = _G // _NS


def _sc_segment_sums(x, batch_i32):
    mesh = plsc.VectorSubcoreMesh(
        core_axis_name="c", subcore_axis_name="s",
        num_cores=_NC, num_subcores=_NS)

    @functools.partial(
        pl.kernel,
        out_type=(
            jax.ShapeDtypeStruct((_NC, _G, _D), jnp.float32),
            jax.ShapeDtypeStruct((_NC, _G, _D), jnp.float32),
        ),
        mesh=mesh,
        scratch_types=[
            pltpu.VMEM((2, _BLK, _D), jnp.float32),
            pltpu.VMEM((2, _BLK), jnp.int32),
            pltpu.VMEM((_TAIL, _D), jnp.float32),
            pltpu.VMEM((_TAIL,), jnp.int32),
            pltpu.VMEM((_BLK, _D), jnp.float32),
            pltpu.VMEM((_GR, _D), jnp.float32),
            pltpu.VMEM_SHARED((_G, _D), jnp.float32),
            pltpu.VMEM_SHARED((_G, _D), jnp.float32),
            pltpu.SemaphoreType.DMA,
            pltpu.SemaphoreType.DMA,
            pltpu.SemaphoreType.DMA,
            pltpu.SemaphoreType.DMA,
            pltpu.SemaphoreType.DMA,
        ],
    )
    def k(x_hbm, b_hbm, sums_out, cnt_out,
          xb, idx, xbt, idxt, ones_v, zb, acc, cacc,
          lx0, li0, lx1, li1, ssem):
        cid = lax.axis_index("c")
        sid = lax.axis_index("s")
        wid = sid * _NC + cid
        lsem = ((lx0, li0), (lx1, li1))

        zeros16 = jnp.zeros((16,), jnp.float32)
        ones16 = jnp.ones((16,), jnp.float32)
        for i in range(_GR):
            for j in range(_D // 16):
                zb[i, pl.ds(j * 16, 16)] = zeros16
        for i in range(_BLK):
            for j in range(_D // 16):
                ones_v[i, pl.ds(j * 16, 16)] = ones16

        pltpu.sync_copy(zb, acc.at[pl.ds(sid * _GR, _GR)])
        pltpu.sync_copy(zb, cacc.at[pl.ds(sid * _GR, _GR)])
        plsc.subcore_barrier()

        def issue_loads(it, p):
            base = (wid + it * _NW) * _BLK
            dx = pltpu.async_copy(x_hbm.at[pl.ds(base, _BLK)],
                                  xb.at[p], lsem[p][0])
            di = pltpu.async_copy(b_hbm.at[pl.ds(base, _BLK)],
                                  idx.at[p], lsem[p][1])
            return dx, di

        pend_ld = issue_loads(0, 0)
        for it in range(_STEADY):
            p = it & 1
            for dsc in pend_ld:
                dsc.wait()
            if it + 1 < _STEADY:
                pend_ld = issue_loads(it + 1, 1 - p)
            da = pltpu.async_copy(xb.at[p], acc.at[idx.at[p]], ssem,
                                  add=True)
            pltpu.sync_copy(ones_v, cacc.at[idx.at[p]], add=True)
            da.wait()

        for it in range(_STEADY, _ITERS):
            blk = wid + it * _NW

            @pl.when(blk < _NFULL)
            def _():
                base = blk * _BLK
                pltpu.sync_copy(b_hbm.at[pl.ds(base, _BLK)], idx.at[0])
                pltpu.sync_copy(x_hbm.at[pl.ds(base, _BLK)], xb.at[0])
                pltpu.sync_copy(xb.at[0], acc.at[idx.at[0]], add=True)
                pltpu.sync_copy(ones_v, cacc.at[idx.at[0]], add=True)

            if _TAIL:
                @pl.when(blk == _NFULL)
                def _():
                    base = _NFULL * _BLK
                    pltpu.sync_copy(b_hbm.at[pl.ds(base, _TAIL)], idxt)
                    pltpu.sync_copy(x_hbm.at[pl.ds(base, _TAIL)], xbt)
                    pltpu.sync_copy(xbt, acc.at[idxt], add=True)
                    pltpu.sync_copy(ones_v.at[pl.ds(0, _TAIL)],
                                    cacc.at[idxt], add=True)

        plsc.subcore_barrier()

        @pl.when(sid == 0)
        def _():
            pltpu.sync_copy(acc, sums_out.at[cid])
            pltpu.sync_copy(cacc, cnt_out.at[cid])

    return k(x, batch_i32)


def _mlp(u, sums2, cnt2, w1u, w1x, b1, w2, b2):
    def body(u_ref, s_ref, c_ref, w1u_ref, w1x_ref, b1_ref, w2_ref, b2_ref,
             o_ref):
        sums = s_ref[0] + s_ref[1]
        cnt = c_ref[0, :, 0:1] + c_ref[1, :, 0:1]
        mean = sums / jnp.maximum(cnt, 1.0)
        h = (jnp.dot(u_ref[...], w1u_ref[...],
                     preferred_element_type=jnp.float32)
             + jnp.dot(mean, w1x_ref[...], preferred_element_type=jnp.float32)
             + b1_ref[...])
        h = jnp.maximum(h, 0.0)
        o_ref[...] = (jnp.dot(h, w2_ref[...],
                              preferred_element_type=jnp.float32)
                      + b2_ref[...])

    return pl.pallas_call(
        body,
        out_shape=jax.ShapeDtypeStruct((_G, 32), jnp.float32),
    )(u, sums2, cnt2, w1u, w1x, b1, w2, b2)


def kernel(x, edge_index, edge_attr, u, batch, W1, b1, W2, b2):
    del edge_index, edge_attr
    b32 = batch.astype(jnp.int32)
    sums2, cnt2 = _sc_segment_sums(x, b32)
    n_glob = u.shape[1]
    return _mlp(u, sums2, cnt2, W1[:n_glob], W1[n_glob:],
                b1.reshape(1, -1), W2, b2.reshape(1, -1))


# speedup vs baseline: 7.7111x; 1.2010x over previous
import functools

import jax
import jax.numpy as jnp
from jax import lax
from jax.experimental import pallas as pl
from jax.experimental.pallas import tpu as pltpu
from jax.experimental.pallas import tpu_sc as plsc

_NC, _NS = 2, 16
_NW = _NC * _NS
_N = 100000
_D = 128
_G = 256
_BLK = 128
_NFULL = _N // _BLK
_TAIL = _N - _NFULL * _BLK
_NBLK = _NFULL + (1 if _TAIL else 0)
_ITERS = -(-_NBLK // _NW)
_STEADY = _NFULL // _NW
_GR = _G // _NS


def _sc_segment_sums(x, batch_i32):
    mesh = plsc.VectorSubcoreMesh(
        core_axis_name="c", subcore_axis_name="s",
        num_cores=_NC, num_subcores=_NS)

    @functools.partial(
        pl.kernel,
        out_type=(
            jax.ShapeDtypeStruct((_NC, _G, _D), jnp.float32),
            jax.ShapeDtypeStruct((_NC, _NS, _G), jnp.float32),
        ),
        mesh=mesh,
        compiler_params=pltpu.CompilerParams(needs_layout_passes=False),
        scratch_types=[
            pltpu.VMEM((2, _BLK, _D), jnp.float32),
            pltpu.VMEM((2, _BLK), jnp.int32),
            pltpu.VMEM((_TAIL, _D), jnp.float32),
            pltpu.VMEM((1, _TAIL), jnp.int32),
            pltpu.VMEM((_G,), jnp.float32),
            pltpu.VMEM((_GR, _D), jnp.float32),
            pltpu.VMEM_SHARED((_G, _D), jnp.float32),
            pltpu.SemaphoreType.DMA,
            pltpu.SemaphoreType.DMA,
            pltpu.SemaphoreType.DMA,
            pltpu.SemaphoreType.DMA,
            pltpu.SemaphoreType.DMA,
        ],
    )
    def k(x_hbm, b_hbm, sums_out, cnt_out,
          xb, idx, xbt, idxt, hist, zb, acc,
          lx0, li0, lx1, li1, ssem):
        cid = lax.axis_index("c")
        sid = lax.axis_index("s")
        wid = sid * _NC + cid
        lsem = ((lx0, li0), (lx1, li1))

        zeros16 = jnp.zeros((16,), jnp.float32)
        for i in range(_GR):
            for j in range(_D // 16):
                zb[i, pl.ds(j * 16, 16)] = zeros16
        for i in range(_G // 16):
            hist[pl.ds(i * 16, 16)] = zeros16

        pltpu.sync_copy(zb, acc.at[pl.ds(sid * _GR, _GR)])
        plsc.subcore_barrier()

        lane = lax.iota(jnp.int32, 16)

        def count_ids(idx2d, row, nrows):
            s_lo = idx2d[row, pl.ds(0, 16)][0]
            s_hi = idx2d[row, pl.ds(16 * (nrows - 1), 16)][15]

            def step(i, s_lo_c):
                s = s_lo_c + i
                c = jnp.zeros((16,), jnp.int32)
                for j in range(nrows):
                    v = idx2d[row, pl.ds(j * 16, 16)]
                    c = c + plsc.all_reduce_population_count(v == s)
                base = (s // 16) * 16
                h = hist[pl.ds(base, 16)]
                hist[pl.ds(base, 16)] = h + jnp.where(
                    lane == s % 16, c.astype(jnp.float32), 0.0)
                return s_lo_c

            lax.fori_loop(0, s_hi - s_lo + 1, step, s_lo)

        def issue_loads(it, p):
            base = (wid + it * _NW) * _BLK
            dx = pltpu.async_copy(x_hbm.at[pl.ds(base, _BLK)],
                                  xb.at[p], lsem[p][0])
            di = pltpu.async_copy(b_hbm.at[pl.ds(base, _BLK)],
                                  idx.at[p], lsem[p][1])
            return dx, di

        pend_ld = issue_loads(0, 0)
        for it in range(_STEADY):
            p = it & 1
            for dsc in pend_ld:
                dsc.wait()
            if it + 1 < _STEADY:
                pend_ld = issue_loads(it + 1, 1 - p)
            da = pltpu.async_copy(xb.at[p], acc.at[idx.at[p]], ssem,
                                  add=True)
            count_ids(idx, p, _BLK // 16)
            da.wait()

        for it in range(_STEADY, _ITERS):
            blk = wid + it * _NW

            @pl.when(blk < _NFULL)
            def _():
                base = blk * _BLK
                pltpu.sync_copy(b_hbm.at[pl.ds(base, _BLK)], idx.at[0])
                pltpu.sync_copy(x_hbm.at[pl.ds(base, _BLK)], xb.at[0])
                pltpu.sync_copy(xb.at[0], acc.at[idx.at[0]], add=True)
                count_ids(idx, 0, _BLK // 16)

            if _TAIL:
                @pl.when(blk == _NFULL)
                def _():
                    base = _NFULL * _BLK
                    pltpu.sync_copy(b_hbm.at[pl.ds(base, _TAIL)], idxt.at[0])
                    pltpu.sync_copy(x_hbm.at[pl.ds(base, _TAIL)], xbt)
                    pltpu.sync_copy(xbt, acc.at[idxt.at[0]], add=True)
                    count_ids(idxt, 0, _TAIL // 16)

        pltpu.sync_copy(hist, cnt_out.at[cid, sid])
        plsc.subcore_barrier()

        @pl.when(sid == 0)
        def _():
            pltpu.sync_copy(acc, sums_out.at[cid])

    return k(x, batch_i32)


def _mlp(u, sums2, cnt2, w1u, w1x, b1, w2, b2):
    def body(u_ref, s_ref, c_ref, w1u_ref, w1x_ref, b1_ref, w2_ref, b2_ref,
             o_ref):
        sums = s_ref[0] + s_ref[1]
        cnt = jnp.sum(c_ref[...], axis=(0, 1))[:, None]
        mean = sums / jnp.maximum(cnt, 1.0)
        h = (jnp.dot(u_ref[...], w1u_ref[...],
                     preferred_element_type=jnp.float32)
             + jnp.dot(mean, w1x_ref[...], preferred_element_type=jnp.float32)
             + b1_ref[...])
        h = jnp.maximum(h, 0.0)
        o_ref[...] = (jnp.dot(h, w2_ref[...],
                              preferred_element_type=jnp.float32)
                      + b2_ref[...])

    return pl.pallas_call(
        body,
        out_shape=jax.ShapeDtypeStruct((_G, 32), jnp.float32),
    )(u, sums2, cnt2, w1u, w1x, b1, w2, b2)


def kernel(x, edge_index, edge_attr, u, batch, W1, b1, W2, b2):
    del edge_index, edge_attr
    b32 = batch.astype(jnp.int32)
    sums2, cnt2 = _sc_segment_sums(x, b32)
    n_glob = u.shape[1]
    return _mlp(u, sums2, cnt2, W1[:n_glob], W1[n_glob:],
                b1.reshape(1, -1), W2, b2.reshape(1, -1))
